# Initial kernel scaffold; baseline (speedup 1.0000x reference)
import functools

import jax
import jax.numpy as jnp
from jax import lax
from jax.experimental import pallas as pl
from jax.experimental.pallas import tpu as pltpu
from jax.experimental.pallas import tpu_sc as plsc

_N = 10000
_E = 320000
_DIN = 128
_DH = 32

_NC = 2
_NS = 16
_NW = _NC * _NS
_IDXW = 128
_ROWS_PER_WID = 80
_EROWS = _NW * _ROWS_PER_WID
_EPAD = _EROWS * _IDXW
_NODE_ROWS = _N // _NS
_DEGW = 16

_BLK = 1000
_GRID = _N // _BLK


def _mesh():
    return plsc.VectorSubcoreMesh(core_axis_name="c", subcore_axis_name="s")


@functools.lru_cache(maxsize=None)
def _sc_deg_kernel():
    @functools.partial(
        pl.kernel,
        out_type=jax.ShapeDtypeStruct((_NC, _N, _DEGW), jnp.float32),
        mesh=_mesh(),
        scratch_types=[
            pltpu.VMEM_SHARED((_N + 1, _DEGW), jnp.float32),
            pltpu.VMEM((_ROWS_PER_WID, _IDXW), jnp.int32),
            pltpu.VMEM((_IDXW, _DEGW), jnp.float32),
        ],
    )
    def deg_kernel(dst_hbm, z_hbm, ones_hbm, out_hbm, dacc, didx, ones_v):
        c = lax.axis_index("c")
        s = lax.axis_index("s")
        wid = c * _NS + s
        pltpu.sync_copy(dst_hbm.at[pl.ds(wid * _ROWS_PER_WID, _ROWS_PER_WID)], didx)
        pltpu.sync_copy(ones_hbm, ones_v)
        r0 = s * _NODE_ROWS
        pltpu.sync_copy(z_hbm.at[pl.ds(r0, _NODE_ROWS)], dacc.at[pl.ds(r0, _NODE_ROWS)])
        plsc.subcore_barrier()

        @pl.loop(0, _ROWS_PER_WID)
        def _(j):
            pltpu.sync_copy(ones_v, dacc.at[didx.at[j]], add=True)

        plsc.subcore_barrier()
        pltpu.sync_copy(dacc.at[pl.ds(r0, _NODE_ROWS)],
                        out_hbm.at[c].at[pl.ds(r0, _NODE_ROWS)])

    return deg_kernel


@functools.lru_cache(maxsize=None)
def _sc_agg_kernel():
    @functools.partial(
        pl.kernel,
        out_type=jax.ShapeDtypeStruct((_NC, _N, _DH), jnp.float32),
        mesh=_mesh(),
        scratch_types=[
            pltpu.VMEM_SHARED((_N + 1, _DH), jnp.float32),
            pltpu.VMEM((_ROWS_PER_WID, _IDXW), jnp.int32),
            pltpu.VMEM((_ROWS_PER_WID, _IDXW), jnp.int32),
            pltpu.VMEM((_IDXW, _DH), jnp.float32),
            pltpu.VMEM((_IDXW, _DH), jnp.float32),
            pltpu.SemaphoreType.DMA,
            pltpu.SemaphoreType.DMA,
        ],
    )
    def agg_kernel(g_hbm, src_hbm, dst_hbm, z_hbm, out_hbm,
                   acc, sidx, didx, rows_a, rows_b, sem_a, sem_b):
        c = lax.axis_index("c")
        s = lax.axis_index("s")
        wid = c * _NS + s
        pltpu.sync_copy(src_hbm.at[pl.ds(wid * _ROWS_PER_WID, _ROWS_PER_WID)], sidx)
        pltpu.sync_copy(dst_hbm.at[pl.ds(wid * _ROWS_PER_WID, _ROWS_PER_WID)], didx)
        r0 = s * _NODE_ROWS
        pltpu.sync_copy(z_hbm.at[pl.ds(r0, _NODE_ROWS)], acc.at[pl.ds(r0, _NODE_ROWS)])
        plsc.subcore_barrier()

        def gather_start(j, buf, sem):
            pltpu.make_async_copy(g_hbm.at[sidx.at[j]], buf, sem).start()

        def gather_wait(j, buf, sem):
            pltpu.make_async_copy(g_hbm.at[sidx.at[j]], buf, sem).wait()

        gather_start(0, rows_a, sem_a)

        @pl.loop(0, _ROWS_PER_WID, step=2)
        def _(j):
            gather_wait(j, rows_a, sem_a)
            gather_start(j + 1, rows_b, sem_b)
            pltpu.sync_copy(rows_a, acc.at[didx.at[j]], add=True)
            gather_wait(j + 1, rows_b, sem_b)

            @pl.when(j + 2 < _ROWS_PER_WID)
            def _():
                gather_start(j + 2, rows_a, sem_a)

            pltpu.sync_copy(rows_b, acc.at[didx.at[j + 1]], add=True)

        plsc.subcore_barrier()
        pltpu.sync_copy(acc.at[pl.ds(r0, _NODE_ROWS)],
                        out_hbm.at[c].at[pl.ds(r0, _NODE_ROWS)])

    return agg_kernel


def _mm_body(x_ref, w_ref, o_ref):
    o_ref[...] = jnp.dot(x_ref[...], w_ref[...],
                         preferred_element_type=jnp.float32,
                         precision=lax.Precision.HIGHEST)


def _tc_matmul(x, w):
    n, k = x.shape
    m = w.shape[1]
    return pl.pallas_call(
        _mm_body,
        grid=(_GRID,),
        in_specs=[pl.BlockSpec((_BLK, k), lambda i: (i, 0)),
                  pl.BlockSpec((k, m), lambda i: (0, 0))],
        out_specs=pl.BlockSpec((_BLK, m), lambda i: (i, 0)),
        out_shape=jax.ShapeDtypeStruct((n, m), jnp.float32),
    )(x, w)


def _prescale_body(d0_ref, d1_ref, h_ref, g_ref, dinv_ref):
    deg = d0_ref[:, 0:1] + d1_ref[:, 0:1] + 1.0
    dinv = lax.rsqrt(deg)
    dinv_ref[...] = dinv
    g_ref[...] = h_ref[...] * dinv


def _tc_prescale(d0, d1, h):
    return pl.pallas_call(
        _prescale_body,
        grid=(_GRID,),
        in_specs=[pl.BlockSpec((_BLK, _DEGW), lambda i: (i, 0)),
                  pl.BlockSpec((_BLK, _DEGW), lambda i: (i, 0)),
                  pl.BlockSpec((_BLK, _DH), lambda i: (i, 0))],
        out_specs=[pl.BlockSpec((_BLK, _DH), lambda i: (i, 0)),
                   pl.BlockSpec((_BLK, 1), lambda i: (i, 0))],
        out_shape=[jax.ShapeDtypeStruct((_N, _DH), jnp.float32),
                   jax.ShapeDtypeStruct((_N, 1), jnp.float32)],
    )(d0, d1, h)


def _mid_body(a0_ref, a1_ref, g_ref, dinv_ref, b_ref, w_ref, o_ref):
    dv = dinv_ref[...]
    t = (a0_ref[...] + a1_ref[...] + g_ref[...]) * dv + b_ref[...]
    t = jnp.maximum(t, 0.0)
    o_ref[...] = jnp.dot(t, w_ref[...],
                         preferred_element_type=jnp.float32,
                         precision=lax.Precision.HIGHEST) * dv


def _tc_mid(a0, a1, g, dinv, b, w):
    return pl.pallas_call(
        _mid_body,
        grid=(_GRID,),
        in_specs=[pl.BlockSpec((_BLK, _DH), lambda i: (i, 0)),
                  pl.BlockSpec((_BLK, _DH), lambda i: (i, 0)),
                  pl.BlockSpec((_BLK, _DH), lambda i: (i, 0)),
                  pl.BlockSpec((_BLK, 1), lambda i: (i, 0)),
                  pl.BlockSpec((1, _DH), lambda i: (0, 0)),
                  pl.BlockSpec((_DH, _DH), lambda i: (0, 0))],
        out_specs=pl.BlockSpec((_BLK, _DH), lambda i: (i, 0)),
        out_shape=jax.ShapeDtypeStruct((_N, _DH), jnp.float32),
    )(a0, a1, g, dinv, b, w)


def _fin_body(a0_ref, a1_ref, g_ref, dinv_ref, b_ref, wl_ref, bl_ref, o_ref):
    t = (a0_ref[...] + a1_ref[...] + g_ref[...]) * dinv_ref[...] + b_ref[...]
    t = jnp.maximum(t, 0.0)
    o_ref[...] = jnp.dot(t, wl_ref[...],
                         preferred_element_type=jnp.float32,
                         precision=lax.Precision.HIGHEST) + bl_ref[...]


def _tc_fin(a0, a1, g, dinv, b, wl, bl):
    return pl.pallas_call(
        _fin_body,
        grid=(_GRID,),
        in_specs=[pl.BlockSpec((_BLK, _DH), lambda i: (i, 0)),
                  pl.BlockSpec((_BLK, _DH), lambda i: (i, 0)),
                  pl.BlockSpec((_BLK, _DH), lambda i: (i, 0)),
                  pl.BlockSpec((_BLK, 1), lambda i: (i, 0)),
                  pl.BlockSpec((1, _DH), lambda i: (0, 0)),
                  pl.BlockSpec((_DH, 1), lambda i: (0, 0)),
                  pl.BlockSpec((1, 1), lambda i: (0, 0))],
        out_specs=pl.BlockSpec((_BLK, 1), lambda i: (i, 0)),
        out_shape=jax.ShapeDtypeStruct((_N, 1), jnp.float32),
    )(a0, a1, g, dinv, b, wl, bl)


def kernel(x, edge_index, W1, b1, W2, b2, Wl, bl):
    f32 = jnp.float32
    src = edge_index[0].astype(jnp.int32)
    dst = edge_index[1].astype(jnp.int32)
    sent = jnp.full((_EPAD - _E,), _N, jnp.int32)
    srcp = jnp.concatenate([src, sent]).reshape(_EROWS, _IDXW)
    dstp = jnp.concatenate([dst, sent]).reshape(_EROWS, _IDXW)
    z32 = jnp.zeros((_N, _DH), f32)
    z16 = jnp.zeros((_N, _DEGW), f32)
    ones = jnp.ones((_IDXW, _DEGW), f32)
    zrow = jnp.zeros((1, _DH), f32)

    degp = _sc_deg_kernel()(dstp, z16, ones)
    h1 = _tc_matmul(x, W1)
    g1, dinv = _tc_prescale(degp[0], degp[1], h1)
    acc1 = _sc_agg_kernel()(jnp.concatenate([g1, zrow]), srcp, dstp, z32)
    g2 = _tc_mid(acc1[0], acc1[1], g1, dinv, b1.reshape(1, _DH), W2)
    acc2 = _sc_agg_kernel()(jnp.concatenate([g2, zrow]), srcp, dstp, z32)
    return _tc_fin(acc2[0], acc2[1], g2, dinv, b2.reshape(1, _DH),
                   Wl, bl.reshape(1, 1))


# speedup vs baseline: 21.3408x; 21.3408x over previous
import functools

import jax
import jax.numpy as jnp
from jax import lax
from jax.experimental import pallas as pl
from jax.experimental.pallas import tpu as pltpu
from jax.experimental.pallas import tpu_sc as plsc

_N = 10000
_E = 320000
_DIN = 128
_DH = 32

_NC = 2
_NS = 16
_NW = _NC * _NS
_IDXW = 128
_ROWS_PER_WID = 80
_EROWS = _NW * _ROWS_PER_WID
_EPAD = _EROWS * _IDXW
_NP = 10240
_NODE_ROWS = _NP // _NS
_DEGW = 16

_BLK = 1000
_GRID = _N // _BLK


def _mesh():
    return plsc.VectorSubcoreMesh(core_axis_name="c", subcore_axis_name="s")


_SC_PARAMS = pltpu.CompilerParams(use_tc_tiling_on_sc=False)


@functools.lru_cache(maxsize=None)
def _sc_deg_kernel():
    @functools.partial(
        pl.kernel,
        out_type=jax.ShapeDtypeStruct((_NC, _NP, _DEGW), jnp.float32),
        mesh=_mesh(),
        compiler_params=_SC_PARAMS,
        scratch_types=[
            pltpu.VMEM_SHARED((_NP, _DEGW), jnp.float32),
            pltpu.VMEM((_ROWS_PER_WID, _IDXW), jnp.int32),
            pltpu.VMEM((_IDXW, _DEGW), jnp.float32),
        ],
    )
    def deg_kernel(dst_hbm, z_hbm, ones_hbm, out_hbm, dacc, didx, ones_v):
        c = lax.axis_index("c")
        s = lax.axis_index("s")
        wid = c * _NS + s
        pltpu.sync_copy(dst_hbm.at[pl.ds(wid * _ROWS_PER_WID, _ROWS_PER_WID)], didx)
        pltpu.sync_copy(ones_hbm, ones_v)
        r0 = s * _NODE_ROWS
        pltpu.sync_copy(z_hbm.at[pl.ds(r0, _NODE_ROWS)], dacc.at[pl.ds(r0, _NODE_ROWS)])
        plsc.subcore_barrier()

        @pl.loop(0, _ROWS_PER_WID)
        def _(j):
            pltpu.sync_copy(ones_v, dacc.at[didx.at[j]], add=True)

        plsc.subcore_barrier()
        pltpu.sync_copy(dacc.at[pl.ds(r0, _NODE_ROWS)],
                        out_hbm.at[c].at[pl.ds(r0, _NODE_ROWS)])

    return deg_kernel


@functools.lru_cache(maxsize=None)
def _sc_agg_kernel():
    @functools.partial(
        pl.kernel,
        out_type=jax.ShapeDtypeStruct((_NC, _NP, _DH), jnp.float32),
        mesh=_mesh(),
        compiler_params=_SC_PARAMS,
        scratch_types=[
            pltpu.VMEM_SHARED((_NP, _DH), jnp.float32),
            pltpu.VMEM((_ROWS_PER_WID, _IDXW), jnp.int32),
            pltpu.VMEM((_ROWS_PER_WID, _IDXW), jnp.int32),
            pltpu.VMEM((_IDXW, _DH), jnp.float32),
            pltpu.VMEM((_IDXW, _DH), jnp.float32),
            pltpu.SemaphoreType.DMA,
            pltpu.SemaphoreType.DMA,
        ],
    )
    def agg_kernel(g_hbm, src_hbm, dst_hbm, z_hbm, out_hbm,
                   acc, sidx, didx, rows_a, rows_b, sem_a, sem_b):
        c = lax.axis_index("c")
        s = lax.axis_index("s")
        wid = c * _NS + s
        pltpu.sync_copy(src_hbm.at[pl.ds(wid * _ROWS_PER_WID, _ROWS_PER_WID)], sidx)
        pltpu.sync_copy(dst_hbm.at[pl.ds(wid * _ROWS_PER_WID, _ROWS_PER_WID)], didx)
        r0 = s * _NODE_ROWS
        pltpu.sync_copy(z_hbm.at[pl.ds(r0, _NODE_ROWS)], acc.at[pl.ds(r0, _NODE_ROWS)])
        plsc.subcore_barrier()

        def gather_start(j, buf, sem):
            pltpu.make_async_copy(g_hbm.at[sidx.at[j]], buf, sem).start()

        def gather_wait(j, buf, sem):
            pltpu.make_async_copy(g_hbm.at[sidx.at[j]], buf, sem).wait()

        gather_start(0, rows_a, sem_a)

        @pl.loop(0, _ROWS_PER_WID, step=2)
        def _(j):
            gather_wait(j, rows_a, sem_a)
            gather_start(j + 1, rows_b, sem_b)
            pltpu.sync_copy(rows_a, acc.at[didx.at[j]], add=True)
            gather_wait(j + 1, rows_b, sem_b)

            @pl.when(j + 2 < _ROWS_PER_WID)
            def _():
                gather_start(j + 2, rows_a, sem_a)

            pltpu.sync_copy(rows_b, acc.at[didx.at[j + 1]], add=True)

        plsc.subcore_barrier()
        pltpu.sync_copy(acc.at[pl.ds(r0, _NODE_ROWS)],
                        out_hbm.at[c].at[pl.ds(r0, _NODE_ROWS)])

    return agg_kernel


def _mm_body(x_ref, w_ref, o_ref):
    o_ref[...] = jnp.dot(x_ref[...], w_ref[...], preferred_element_type=jnp.float32)


def _tc_matmul(x, w):
    n, k = x.shape
    m = w.shape[1]
    return pl.pallas_call(
        _mm_body,
        grid=(_GRID,),
        in_specs=[pl.BlockSpec((_BLK, k), lambda i: (i, 0)),
                  pl.BlockSpec((k, m), lambda i: (0, 0))],
        out_specs=pl.BlockSpec((_BLK, m), lambda i: (i, 0)),
        out_shape=jax.ShapeDtypeStruct((n, m), jnp.float32),
    )(x, w)


def _prescale_body(d0_ref, d1_ref, h_ref, g_ref, dinv_ref):
    deg = d0_ref[:, 0:1] + d1_ref[:, 0:1] + 1.0
    dinv = lax.rsqrt(deg)
    dinv_ref[...] = dinv
    g_ref[...] = h_ref[...] * dinv


def _tc_prescale(d0, d1, h):
    return pl.pallas_call(
        _prescale_body,
        grid=(_GRID,),
        in_specs=[pl.BlockSpec((_BLK, _DEGW), lambda i: (i, 0)),
                  pl.BlockSpec((_BLK, _DEGW), lambda i: (i, 0)),
                  pl.BlockSpec((_BLK, _DH), lambda i: (i, 0))],
        out_specs=[pl.BlockSpec((_BLK, _DH), lambda i: (i, 0)),
                   pl.BlockSpec((_BLK, 1), lambda i: (i, 0))],
        out_shape=[jax.ShapeDtypeStruct((_N, _DH), jnp.float32),
                   jax.ShapeDtypeStruct((_N, 1), jnp.float32)],
    )(d0, d1, h)


def _mid_body(a0_ref, a1_ref, g_ref, dinv_ref, b_ref, w_ref, o_ref):
    dv = dinv_ref[...]
    t = (a0_ref[...] + a1_ref[...] + g_ref[...]) * dv + b_ref[...]
    t = jnp.maximum(t, 0.0)
    o_ref[...] = jnp.dot(t, w_ref[...], preferred_element_type=jnp.float32) * dv


def _tc_mid(a0, a1, g, dinv, b, w):
    return pl.pallas_call(
        _mid_body,
        grid=(_GRID,),
        in_specs=[pl.BlockSpec((_BLK, _DH), lambda i: (i, 0)),
                  pl.BlockSpec((_BLK, _DH), lambda i: (i, 0)),
                  pl.BlockSpec((_BLK, _DH), lambda i: (i, 0)),
                  pl.BlockSpec((_BLK, 1), lambda i: (i, 0)),
                  pl.BlockSpec((1, _DH), lambda i: (0, 0)),
                  pl.BlockSpec((_DH, _DH), lambda i: (0, 0))],
        out_specs=pl.BlockSpec((_BLK, _DH), lambda i: (i, 0)),
        out_shape=jax.ShapeDtypeStruct((_N, _DH), jnp.float32),
    )(a0, a1, g, dinv, b, w)


def _fin_body(a0_ref, a1_ref, g_ref, dinv_ref, b_ref, wl_ref, bl_ref, o_ref):
    t = (a0_ref[...] + a1_ref[...] + g_ref[...]) * dinv_ref[...] + b_ref[...]
    t = jnp.maximum(t, 0.0)
    o_ref[...] = jnp.dot(t, wl_ref[...], preferred_element_type=jnp.float32) + bl_ref[...]


def _tc_fin(a0, a1, g, dinv, b, wl, bl):
    return pl.pallas_call(
        _fin_body,
        grid=(_GRID,),
        in_specs=[pl.BlockSpec((_BLK, _DH), lambda i: (i, 0)),
                  pl.BlockSpec((_BLK, _DH), lambda i: (i, 0)),
                  pl.BlockSpec((_BLK, _DH), lambda i: (i, 0)),
                  pl.BlockSpec((_BLK, 1), lambda i: (i, 0)),
                  pl.BlockSpec((1, _DH), lambda i: (0, 0)),
                  pl.BlockSpec((_DH, 1), lambda i: (0, 0)),
                  pl.BlockSpec((1, 1), lambda i: (0, 0))],
        out_specs=pl.BlockSpec((_BLK, 1), lambda i: (i, 0)),
        out_shape=jax.ShapeDtypeStruct((_N, 1), jnp.float32),
    )(a0, a1, g, dinv, b, wl, bl)


def kernel(x, edge_index, W1, b1, W2, b2, Wl, bl):
    f32 = jnp.float32
    src = edge_index[0].astype(jnp.int32)
    dst = edge_index[1].astype(jnp.int32)
    sent = jnp.full((_EPAD - _E,), _N, jnp.int32)
    srcp = jnp.concatenate([src, sent]).reshape(_EROWS, _IDXW)
    dstp = jnp.concatenate([dst, sent]).reshape(_EROWS, _IDXW)
    z32 = jnp.zeros((_NP, _DH), f32)
    z16 = jnp.zeros((_NP, _DEGW), f32)
    ones = jnp.ones((_IDXW, _DEGW), f32)
    zpad = jnp.zeros((_NP - _N, _DH), f32)

    degp = _sc_deg_kernel()(dstp, z16, ones)
    h1 = _tc_matmul(x, W1)
    g1, dinv = _tc_prescale(degp[0, :_N], degp[1, :_N], h1)
    acc1 = _sc_agg_kernel()(jnp.concatenate([g1, zpad]), srcp, dstp, z32)
    g2 = _tc_mid(acc1[0, :_N], acc1[1, :_N], g1, dinv, b1.reshape(1, _DH), W2)
    acc2 = _sc_agg_kernel()(jnp.concatenate([g2, zpad]), srcp, dstp, z32)
    return _tc_fin(acc2[0, :_N], acc2[1, :_N], g2, dinv, b2.reshape(1, _DH),
                   Wl, bl.reshape(1, 1))


# speedup vs baseline: 40.1168x; 1.8798x over previous
import functools

import jax
import jax.numpy as jnp
from jax import lax
from jax.experimental import pallas as pl
from jax.experimental.pallas import tpu as pltpu
from jax.experimental.pallas import tpu_sc as plsc

_N = 10000
_E = 320000
_DIN = 128
_DH = 32

_NC = 2
_NS = 16
_NW = _NC * _NS
_IDXW = 128
_ROWS_PER_WID = 80
_EROWS = _NW * _ROWS_PER_WID
_EPAD = _EROWS * _IDXW
_NP = 10240
_NODE_ROWS = _NP // _NS
_DEGW = 16

_BLK = 1000
_GRID = _N // _BLK


def _mesh():
    return plsc.VectorSubcoreMesh(core_axis_name="c", subcore_axis_name="s")


_SC_PARAMS = pltpu.CompilerParams(use_tc_tiling_on_sc=False)


@functools.lru_cache(maxsize=None)
def _sc_deg_kernel():
    @functools.partial(
        pl.kernel,
        out_type=jax.ShapeDtypeStruct((_NC, _NP, _DEGW), jnp.float32),
        mesh=_mesh(),
        compiler_params=_SC_PARAMS,
        scratch_types=[
            pltpu.VMEM_SHARED((_NP, _DEGW), jnp.float32),
            pltpu.VMEM((_ROWS_PER_WID, _IDXW), jnp.int32),
            pltpu.VMEM((_IDXW, _DEGW), jnp.float32),
        ],
    )
    def deg_kernel(dst_hbm, z_hbm, ones_hbm, out_hbm, dacc, didx, ones_v):
        c = lax.axis_index("c")
        s = lax.axis_index("s")
        wid = c * _NS + s
        pltpu.sync_copy(dst_hbm.at[pl.ds(wid * _ROWS_PER_WID, _ROWS_PER_WID)], didx)
        pltpu.sync_copy(ones_hbm, ones_v)
        r0 = s * _NODE_ROWS
        pltpu.sync_copy(z_hbm.at[pl.ds(r0, _NODE_ROWS)], dacc.at[pl.ds(r0, _NODE_ROWS)])
        plsc.subcore_barrier()

        @pl.loop(0, _ROWS_PER_WID)
        def _(j):
            pltpu.sync_copy(ones_v, dacc.at[didx.at[j]], add=True)

        plsc.subcore_barrier()
        pltpu.sync_copy(dacc.at[pl.ds(r0, _NODE_ROWS)],
                        out_hbm.at[c].at[pl.ds(r0, _NODE_ROWS)])

    return deg_kernel


@functools.lru_cache(maxsize=None)
def _sc_agg_kernel():
    @functools.partial(
        pl.kernel,
        out_type=jax.ShapeDtypeStruct((_NC, _NP, _DH), jnp.float32),
        mesh=_mesh(),
        compiler_params=_SC_PARAMS,
        scratch_types=[
            pltpu.VMEM_SHARED((_NP, _DH), jnp.float32),
            pltpu.VMEM_SHARED((_NP, _DH), jnp.float32),
            pltpu.VMEM((_ROWS_PER_WID, _IDXW), jnp.int32),
            pltpu.VMEM((_ROWS_PER_WID, _IDXW), jnp.int32),
            pltpu.VMEM((_IDXW, _DH), jnp.float32),
            pltpu.VMEM((_IDXW, _DH), jnp.float32),
            pltpu.SemaphoreType.DMA,
            pltpu.SemaphoreType.DMA,
        ],
    )
    def agg_kernel(g_hbm, src_hbm, dst_hbm, z_hbm, out_hbm,
                   acc, gtab, sidx, didx, rows_a, rows_b, sem_a, sem_b):
        c = lax.axis_index("c")
        s = lax.axis_index("s")
        wid = c * _NS + s
        pltpu.sync_copy(src_hbm.at[pl.ds(wid * _ROWS_PER_WID, _ROWS_PER_WID)], sidx)
        pltpu.sync_copy(dst_hbm.at[pl.ds(wid * _ROWS_PER_WID, _ROWS_PER_WID)], didx)
        r0 = s * _NODE_ROWS
        pltpu.sync_copy(g_hbm.at[pl.ds(r0, _NODE_ROWS)], gtab.at[pl.ds(r0, _NODE_ROWS)])
        pltpu.sync_copy(z_hbm.at[pl.ds(r0, _NODE_ROWS)], acc.at[pl.ds(r0, _NODE_ROWS)])
        plsc.subcore_barrier()

        def gather_start(j, buf, sem):
            pltpu.make_async_copy(gtab.at[sidx.at[j]], buf, sem).start()

        def gather_wait(j, buf, sem):
            pltpu.make_async_copy(gtab.at[sidx.at[j]], buf, sem).wait()

        gather_start(0, rows_a, sem_a)

        @pl.loop(0, _ROWS_PER_WID, step=2)
        def _(j):
            gather_wait(j, rows_a, sem_a)
            gather_start(j + 1, rows_b, sem_b)
            pltpu.sync_copy(rows_a, acc.at[didx.at[j]], add=True)
            gather_wait(j + 1, rows_b, sem_b)

            @pl.when(j + 2 < _ROWS_PER_WID)
            def _():
                gather_start(j + 2, rows_a, sem_a)

            pltpu.sync_copy(rows_b, acc.at[didx.at[j + 1]], add=True)

        plsc.subcore_barrier()
        pltpu.sync_copy(acc.at[pl.ds(r0, _NODE_ROWS)],
                        out_hbm.at[c].at[pl.ds(r0, _NODE_ROWS)])

    return agg_kernel


def _mm_body(x_ref, w_ref, o_ref):
    o_ref[...] = jnp.dot(x_ref[...], w_ref[...], preferred_element_type=jnp.float32)


def _tc_matmul(x, w):
    n, k = x.shape
    m = w.shape[1]
    return pl.pallas_call(
        _mm_body,
        grid=(_GRID,),
        in_specs=[pl.BlockSpec((_BLK, k), lambda i: (i, 0)),
                  pl.BlockSpec((k, m), lambda i: (0, 0))],
        out_specs=pl.BlockSpec((_BLK, m), lambda i: (i, 0)),
        out_shape=jax.ShapeDtypeStruct((n, m), jnp.float32),
    )(x, w)


def _prescale_body(d0_ref, d1_ref, h_ref, g_ref, dinv_ref):
    deg = d0_ref[:, 0:1] + d1_ref[:, 0:1] + 1.0
    dinv = lax.rsqrt(deg)
    dinv_ref[...] = dinv
    g_ref[...] = h_ref[...] * dinv


def _tc_prescale(d0, d1, h):
    return pl.pallas_call(
        _prescale_body,
        grid=(_GRID,),
        in_specs=[pl.BlockSpec((_BLK, _DEGW), lambda i: (i, 0)),
                  pl.BlockSpec((_BLK, _DEGW), lambda i: (i, 0)),
                  pl.BlockSpec((_BLK, _DH), lambda i: (i, 0))],
        out_specs=[pl.BlockSpec((_BLK, _DH), lambda i: (i, 0)),
                   pl.BlockSpec((_BLK, 1), lambda i: (i, 0))],
        out_shape=[jax.ShapeDtypeStruct((_N, _DH), jnp.float32),
                   jax.ShapeDtypeStruct((_N, 1), jnp.float32)],
    )(d0, d1, h)


def _mid_body(a0_ref, a1_ref, g_ref, dinv_ref, b_ref, w_ref, o_ref):
    dv = dinv_ref[...]
    t = (a0_ref[...] + a1_ref[...] + g_ref[...]) * dv + b_ref[...]
    t = jnp.maximum(t, 0.0)
    o_ref[...] = jnp.dot(t, w_ref[...], preferred_element_type=jnp.float32) * dv


def _tc_mid(a0, a1, g, dinv, b, w):
    return pl.pallas_call(
        _mid_body,
        grid=(_GRID,),
        in_specs=[pl.BlockSpec((_BLK, _DH), lambda i: (i, 0)),
                  pl.BlockSpec((_BLK, _DH), lambda i: (i, 0)),
                  pl.BlockSpec((_BLK, _DH), lambda i: (i, 0)),
                  pl.BlockSpec((_BLK, 1), lambda i: (i, 0)),
                  pl.BlockSpec((1, _DH), lambda i: (0, 0)),
                  pl.BlockSpec((_DH, _DH), lambda i: (0, 0))],
        out_specs=pl.BlockSpec((_BLK, _DH), lambda i: (i, 0)),
        out_shape=jax.ShapeDtypeStruct((_N, _DH), jnp.float32),
    )(a0, a1, g, dinv, b, w)


def _fin_body(a0_ref, a1_ref, g_ref, dinv_ref, b_ref, wl_ref, bl_ref, o_ref):
    t = (a0_ref[...] + a1_ref[...] + g_ref[...]) * dinv_ref[...] + b_ref[...]
    t = jnp.maximum(t, 0.0)
    o_ref[...] = jnp.dot(t, wl_ref[...], preferred_element_type=jnp.float32) + bl_ref[...]


def _tc_fin(a0, a1, g, dinv, b, wl, bl):
    return pl.pallas_call(
        _fin_body,
        grid=(_GRID,),
        in_specs=[pl.BlockSpec((_BLK, _DH), lambda i: (i, 0)),
                  pl.BlockSpec((_BLK, _DH), lambda i: (i, 0)),
                  pl.BlockSpec((_BLK, _DH), lambda i: (i, 0)),
                  pl.BlockSpec((_BLK, 1), lambda i: (i, 0)),
                  pl.BlockSpec((1, _DH), lambda i: (0, 0)),
                  pl.BlockSpec((_DH, 1), lambda i: (0, 0)),
                  pl.BlockSpec((1, 1), lambda i: (0, 0))],
        out_specs=pl.BlockSpec((_BLK, 1), lambda i: (i, 0)),
        out_shape=jax.ShapeDtypeStruct((_N, 1), jnp.float32),
    )(a0, a1, g, dinv, b, wl, bl)


def kernel(x, edge_index, W1, b1, W2, b2, Wl, bl):
    f32 = jnp.float32
    src = edge_index[0].astype(jnp.int32)
    dst = edge_index[1].astype(jnp.int32)
    sent = jnp.full((_EPAD - _E,), _N, jnp.int32)
    srcp = jnp.concatenate([src, sent]).reshape(_EROWS, _IDXW)
    dstp = jnp.concatenate([dst, sent]).reshape(_EROWS, _IDXW)
    z32 = jnp.zeros((_NP, _DH), f32)
    z16 = jnp.zeros((_NP, _DEGW), f32)
    ones = jnp.ones((_IDXW, _DEGW), f32)
    zpad = jnp.zeros((_NP - _N, _DH), f32)

    degp = _sc_deg_kernel()(dstp, z16, ones)
    h1 = _tc_matmul(x, W1)
    g1, dinv = _tc_prescale(degp[0, :_N], degp[1, :_N], h1)
    acc1 = _sc_agg_kernel()(jnp.concatenate([g1, zpad]), srcp, dstp, z32)
    g2 = _tc_mid(acc1[0, :_N], acc1[1, :_N], g1, dinv, b1.reshape(1, _DH), W2)
    acc2 = _sc_agg_kernel()(jnp.concatenate([g2, zpad]), srcp, dstp, z32)
    return _tc_fin(acc2[0, :_N], acc2[1, :_N], g2, dinv, b2.reshape(1, _DH),
                   Wl, bl.reshape(1, 1))


# speedup vs baseline: 40.6411x; 1.0131x over previous
import functools

import jax
import jax.numpy as jnp
from jax import lax
from jax.experimental import pallas as pl
from jax.experimental.pallas import tpu as pltpu
from jax.experimental.pallas import tpu_sc as plsc

_N = 10000
_E = 320000
_DIN = 128
_DH = 32

_NC = 2
_NS = 16
_NW = _NC * _NS
_IDXW = 128
_ROWS_PER_WID = 80
_EROWS = _NW * _ROWS_PER_WID
_EPAD = _EROWS * _IDXW
_NP = 10240
_NODE_ROWS = _NP // _NS
_DEGW = 16

_BLKP = 1280
_GRIDP = _NP // _BLKP


def _mesh():
    return plsc.VectorSubcoreMesh(core_axis_name="c", subcore_axis_name="s")


_SC_PARAMS = pltpu.CompilerParams(use_tc_tiling_on_sc=False)


@functools.lru_cache(maxsize=None)
def _sc_deg_kernel():
    @functools.partial(
        pl.kernel,
        out_type=jax.ShapeDtypeStruct((_NC, _NP, _DEGW), jnp.float32),
        mesh=_mesh(),
        compiler_params=_SC_PARAMS,
        scratch_types=[
            pltpu.VMEM_SHARED((_NP, _DEGW), jnp.float32),
            pltpu.VMEM((_ROWS_PER_WID, _IDXW), jnp.int32),
            pltpu.VMEM((_IDXW, _DEGW), jnp.float32),
        ],
    )
    def deg_kernel(dst_hbm, z_hbm, ones_hbm, out_hbm, dacc, didx, ones_v):
        c = lax.axis_index("c")
        s = lax.axis_index("s")
        wid = c * _NS + s
        pltpu.sync_copy(dst_hbm.at[pl.ds(wid * _ROWS_PER_WID, _ROWS_PER_WID)], didx)
        pltpu.sync_copy(ones_hbm, ones_v)
        r0 = s * _NODE_ROWS

        @pl.loop(0, _NODE_ROWS, step=_IDXW)
        def _(r):
            pltpu.sync_copy(z_hbm, dacc.at[pl.ds(r0 + r, _IDXW)])

        plsc.subcore_barrier()

        @pl.loop(0, _ROWS_PER_WID)
        def _(j):
            pltpu.sync_copy(ones_v, dacc.at[didx.at[j]], add=True)

        plsc.subcore_barrier()
        pltpu.sync_copy(dacc.at[pl.ds(r0, _NODE_ROWS)],
                        out_hbm.at[c].at[pl.ds(r0, _NODE_ROWS)])

    return deg_kernel


@functools.lru_cache(maxsize=None)
def _sc_agg_kernel():
    @functools.partial(
        pl.kernel,
        out_type=jax.ShapeDtypeStruct((_NC, _NP, _DH), jnp.float32),
        mesh=_mesh(),
        compiler_params=_SC_PARAMS,
        scratch_types=[
            pltpu.VMEM_SHARED((_NP, _DH), jnp.float32),
            pltpu.VMEM_SHARED((_NP, _DH), jnp.float32),
            pltpu.VMEM((_ROWS_PER_WID, _IDXW), jnp.int32),
            pltpu.VMEM((_ROWS_PER_WID, _IDXW), jnp.int32),
            pltpu.VMEM((_IDXW, _DH), jnp.float32),
            pltpu.VMEM((_IDXW, _DH), jnp.float32),
            pltpu.SemaphoreType.DMA,
            pltpu.SemaphoreType.DMA,
        ],
    )
    def agg_kernel(g_hbm, src_hbm, dst_hbm, z_hbm, out_hbm,
                   acc, gtab, sidx, didx, rows_a, rows_b, sem_a, sem_b):
        c = lax.axis_index("c")
        s = lax.axis_index("s")
        wid = c * _NS + s
        pltpu.sync_copy(src_hbm.at[pl.ds(wid * _ROWS_PER_WID, _ROWS_PER_WID)], sidx)
        pltpu.sync_copy(dst_hbm.at[pl.ds(wid * _ROWS_PER_WID, _ROWS_PER_WID)], didx)
        r0 = s * _NODE_ROWS
        pltpu.sync_copy(g_hbm.at[pl.ds(r0, _NODE_ROWS)], gtab.at[pl.ds(r0, _NODE_ROWS)])

        @pl.loop(0, _NODE_ROWS, step=_IDXW)
        def _(r):
            pltpu.sync_copy(z_hbm, acc.at[pl.ds(r0 + r, _IDXW)])

        plsc.subcore_barrier()

        def gather_start(j, buf, sem):
            pltpu.make_async_copy(gtab.at[sidx.at[j]], buf, sem).start()

        def gather_wait(j, buf, sem):
            pltpu.make_async_copy(gtab.at[sidx.at[j]], buf, sem).wait()

        gather_start(0, rows_a, sem_a)

        @pl.loop(0, _ROWS_PER_WID, step=2)
        def _(j):
            gather_wait(j, rows_a, sem_a)
            gather_start(j + 1, rows_b, sem_b)
            pltpu.sync_copy(rows_a, acc.at[didx.at[j]], add=True)
            gather_wait(j + 1, rows_b, sem_b)

            @pl.when(j + 2 < _ROWS_PER_WID)
            def _():
                gather_start(j + 2, rows_a, sem_a)

            pltpu.sync_copy(rows_b, acc.at[didx.at[j + 1]], add=True)

        plsc.subcore_barrier()
        pltpu.sync_copy(acc.at[pl.ds(r0, _NODE_ROWS)],
                        out_hbm.at[c].at[pl.ds(r0, _NODE_ROWS)])

    return agg_kernel


def _mm_body(x_ref, w_ref, o_ref):
    o_ref[...] = jnp.dot(x_ref[...], w_ref[...], preferred_element_type=jnp.float32)


def _tc_matmul(x, w):
    return pl.pallas_call(
        _mm_body,
        grid=(_GRIDP,),
        in_specs=[pl.BlockSpec((_BLKP, _DIN), lambda i: (i, 0)),
                  pl.BlockSpec((_DIN, _DH), lambda i: (0, 0))],
        out_specs=pl.BlockSpec((_BLKP, _DH), lambda i: (i, 0)),
        out_shape=jax.ShapeDtypeStruct((_NP, _DH), jnp.float32),
    )(x, w)


def _prescale_body(d_ref, h_ref, g_ref, dinv_ref):
    deg = d_ref[0, :, 0:1] + d_ref[1, :, 0:1] + 1.0
    dinv = lax.rsqrt(deg)
    dinv_ref[...] = dinv
    g_ref[...] = h_ref[...] * dinv


def _tc_prescale(degp, h):
    return pl.pallas_call(
        _prescale_body,
        grid=(_GRIDP,),
        in_specs=[pl.BlockSpec((_NC, _BLKP, _DEGW), lambda i: (0, i, 0)),
                  pl.BlockSpec((_BLKP, _DH), lambda i: (i, 0))],
        out_specs=[pl.BlockSpec((_BLKP, _DH), lambda i: (i, 0)),
                   pl.BlockSpec((_BLKP, 1), lambda i: (i, 0))],
        out_shape=[jax.ShapeDtypeStruct((_NP, _DH), jnp.float32),
                   jax.ShapeDtypeStruct((_NP, 1), jnp.float32)],
    )(degp, h)


def _mid_body(a_ref, g_ref, dinv_ref, b_ref, w_ref, o_ref):
    dv = dinv_ref[...]
    t = (a_ref[0] + a_ref[1] + g_ref[...]) * dv + b_ref[...]
    t = jnp.maximum(t, 0.0)
    o_ref[...] = jnp.dot(t, w_ref[...], preferred_element_type=jnp.float32) * dv


def _tc_mid(acc, g, dinv, b, w):
    return pl.pallas_call(
        _mid_body,
        grid=(_GRIDP,),
        in_specs=[pl.BlockSpec((_NC, _BLKP, _DH), lambda i: (0, i, 0)),
                  pl.BlockSpec((_BLKP, _DH), lambda i: (i, 0)),
                  pl.BlockSpec((_BLKP, 1), lambda i: (i, 0)),
                  pl.BlockSpec((1, _DH), lambda i: (0, 0)),
                  pl.BlockSpec((_DH, _DH), lambda i: (0, 0))],
        out_specs=pl.BlockSpec((_BLKP, _DH), lambda i: (i, 0)),
        out_shape=jax.ShapeDtypeStruct((_NP, _DH), jnp.float32),
    )(acc, g, dinv, b, w)


def _fin_body(a_ref, g_ref, dinv_ref, b_ref, wl_ref, bl_ref, o_ref):
    t = (a_ref[0] + a_ref[1] + g_ref[...]) * dinv_ref[...] + b_ref[...]
    t = jnp.maximum(t, 0.0)
    o_ref[...] = jnp.dot(t, wl_ref[...], preferred_element_type=jnp.float32) + bl_ref[...]


def _tc_fin(acc, g, dinv, b, wl, bl):
    return pl.pallas_call(
        _fin_body,
        grid=(_GRIDP,),
        in_specs=[pl.BlockSpec((_NC, _BLKP, _DH), lambda i: (0, i, 0)),
                  pl.BlockSpec((_BLKP, _DH), lambda i: (i, 0)),
                  pl.BlockSpec((_BLKP, 1), lambda i: (i, 0)),
                  pl.BlockSpec((1, _DH), lambda i: (0, 0)),
                  pl.BlockSpec((_DH, 1), lambda i: (0, 0)),
                  pl.BlockSpec((1, 1), lambda i: (0, 0))],
        out_specs=pl.BlockSpec((_BLKP, 1), lambda i: (i, 0)),
        out_shape=jax.ShapeDtypeStruct((_NP, 1), jnp.float32),
    )(acc, g, dinv, b, wl, bl)


def kernel(x, edge_index, W1, b1, W2, b2, Wl, bl):
    f32 = jnp.float32
    src = edge_index[0].astype(jnp.int32)
    dst = edge_index[1].astype(jnp.int32)
    sent = jnp.full((_EPAD - _E,), _N, jnp.int32)
    srcp = jnp.concatenate([src, sent]).reshape(_EROWS, _IDXW)
    dstp = jnp.concatenate([dst, sent]).reshape(_EROWS, _IDXW)
    z32 = jnp.zeros((_IDXW, _DH), f32)
    z16 = jnp.zeros((_IDXW, _DEGW), f32)
    ones = jnp.ones((_IDXW, _DEGW), f32)

    degp = _sc_deg_kernel()(dstp, z16, ones)
    h1 = _tc_matmul(x, W1)
    g1, dinv = _tc_prescale(degp, h1)
    acc1 = _sc_agg_kernel()(g1, srcp, dstp, z32)
    g2 = _tc_mid(acc1, g1, dinv, b1.reshape(1, _DH), W2)
    acc2 = _sc_agg_kernel()(g2, srcp, dstp, z32)
    out = _tc_fin(acc2, g2, dinv, b2.reshape(1, _DH), Wl, bl.reshape(1, 1))
    return out[:_N]
